# baseline (speedup 1.0000x reference)
import dataclasses
import functools

import jax
import jax.numpy as jnp
from jax import lax
from jax.experimental import pallas as pl
from jax.experimental.pallas import tpu as pltpu
from jax.experimental.pallas import tpu_sc as plsc

L = 16


def _spline_sc(cpack, wgt, idx, *, B, C, X, Y, interpret=False):
    XY = X * Y
    NC, NS = 2, 16
    NW = NC * NS
    total = B * XY
    ppw = total // NW
    wpb = NW // B
    N = min(512, ppw)
    NH = 0
    nblk = ppw // N
    assert nblk % 2 == 0 and nblk >= 4
    mesh = plsc.VectorSubcoreMesh(
        core_axis_name="c", subcore_axis_name="s",
        num_cores=NC, num_subcores=NS)
    cp = pltpu.CompilerParams()
    if "needs_layout_passes" in pltpu.CompilerParams.__dataclass_fields__:
        cp = dataclasses.replace(cp, needs_layout_passes=False)

    @functools.partial(
        pl.kernel,
        out_type=jax.ShapeDtypeStruct((B, C, XY), jnp.float32),
        mesh=mesh,
        compiler_params=cp,
        interpret=interpret,
        scratch_types=[
            [pltpu.VMEM((4, N), jnp.int32)] * 2,
            [pltpu.VMEM((4, N), jnp.int32)] * 2,
            [[pltpu.VMEM((N,), jnp.int32)] * 16] * 2,
            [[pltpu.VMEM((N,), jnp.int32)] * 16] * 2,
            [pltpu.VMEM((C, N), jnp.float32)] * 2,
            pltpu.VMEM_SHARED((total,), jnp.int32),
            pltpu.SemaphoreType.DMA,
            [pltpu.SemaphoreType.DMA] * 2,
            [pltpu.SemaphoreType.DMA] * 2,
            [pltpu.SemaphoreType.DMA] * 2,
        ],
    )
    def k(cpack_hbm, w_hbm, i_hbm, out_hbm,
          wbuf, ibuf, idxc, gbuf, obuf, ctab, semIn, semG, semGH, semO):
        sid = lax.axis_index("s")
        wid = sid * NC + lax.axis_index("c")
        b = wid // wpb
        wofs = (wid % wpb) * ppw
        bofs = b * XY

        @pl.when(sid == 0)
        def _stage():
            pltpu.sync_copy(cpack_hbm, ctab)

        plsc.subcore_barrier()

        def do_in(m, p):
            p0 = wofs + m * N

            @pl.loop(0, 4)
            def _ld(r):
                pltpu.async_copy(
                    w_hbm.at[r, b, pl.ds(p0, N)], wbuf[p].at[r], semIn)
                pltpu.async_copy(
                    i_hbm.at[r, b, pl.ds(p0, N)], ibuf[p].at[r], semIn)

        def do_build_fire(m, p):
            @pl.loop(0, 8)
            def _dw(r):
                pltpu.make_async_copy(
                    i_hbm.at[0, 0, pl.ds(0, N)], ibuf[p].at[0], semIn
                ).wait()

            @pl.loop(0, N, step=L)
            def _ix(j):
                sl = pl.ds(j, L)
                iqs = [ibuf[p][k, sl] for k in range(4)]
                iys = [iq & jnp.int32(0xFFFF) for iq in iqs]
                for k1 in range(4):
                    row = (iqs[k1] >> 16) * Y + bofs
                    for k2 in range(4):
                        idxc[p][k1 * 4 + k2][sl] = row + iys[k2]

            for cc in range(16):
                if cc < NH:
                    pltpu.async_copy(
                        cpack_hbm.at[idxc[p][cc]], gbuf[p][cc], semGH[p])
                else:
                    pltpu.async_copy(
                        ctab.at[idxc[p][cc]], gbuf[p][cc], semG[p])

        def do_acc(m, p):
            def _reclaim():
                @pl.loop(0, C)
                def _ow(ch):
                    pltpu.make_async_copy(
                        out_hbm.at[0, 0, pl.ds(0, N)], obuf[p].at[0], semO[p]
                    ).wait()

            if isinstance(m, int):
                if m >= 2:
                    _reclaim()
            else:
                pl.when(m >= 2)(_reclaim)

            @pl.loop(0, NH)
            def _gwh(cc):
                pltpu.make_async_copy(
                    i_hbm.at[0, 0, pl.ds(0, N)], gbuf[p][0], semGH[p]
                ).wait()

            @pl.loop(0, 16 - NH)
            def _gw(cc):
                pltpu.make_async_copy(
                    i_hbm.at[0, 0, pl.ds(0, N)], gbuf[p][0], semG[p]
                ).wait()

            @pl.loop(0, N, step=L)
            def _acc(j):
                sl = pl.ds(j, L)
                wq = [wbuf[p][k, sl] for k in range(4)]
                w1 = [plsc.bitcast(q & jnp.int32(-65536), jnp.float32)
                      for q in wq]
                w2 = [plsc.bitcast(q << 16, jnp.float32) for q in wq]
                acc0 = jnp.zeros((L,), jnp.float32)
                acc1 = jnp.zeros((L,), jnp.float32)
                for k1 in range(4):
                    for k2 in range(4):
                        g = gbuf[p][k1 * 4 + k2][sl]
                        lo = plsc.bitcast(g << 16, jnp.float32)
                        hi = plsc.bitcast(g & jnp.int32(-65536), jnp.float32)
                        wp = w1[k1] * w2[k2]
                        acc0 = acc0 + wp * lo
                        acc1 = acc1 + wp * hi
                obuf[p][0, sl] = acc0
                obuf[p][1, sl] = acc1

            p0 = wofs + m * N

            @pl.loop(0, C)
            def _st(ch):
                pltpu.async_copy(
                    obuf[p].at[ch], out_hbm.at[b, ch, pl.ds(p0, N)], semO[p])

        do_in(0, 0)
        do_build_fire(0, 0)
        do_in(1, 1)

        @pl.loop(0, (nblk - 2) // 2)
        def _t(t):
            m = 2 * t + 1
            do_build_fire(m, 1)
            do_acc(m - 1, 0)
            do_in(m + 1, 0)
            do_build_fire(m + 1, 0)
            do_acc(m, 1)
            do_in(m + 2, 1)

        do_build_fire(nblk - 1, 1)
        do_acc(nblk - 2, 0)
        do_acc(nblk - 1, 1)

        for pp in range(2):
            @pl.loop(0, C)
            def _fw(i, pp=pp):
                pltpu.make_async_copy(
                    out_hbm.at[0, 0, pl.ds(0, N)], obuf[0].at[0], semO[pp]
                ).wait()

    return k(cpack, wgt, idx)


def _pack_bf16_pair(hi, lo):
    uh = lax.bitcast_convert_type(hi.astype(jnp.bfloat16), jnp.uint16)
    ul = lax.bitcast_convert_type(lo.astype(jnp.bfloat16), jnp.uint16)
    word = (uh.astype(jnp.uint32) << 16) | ul.astype(jnp.uint32)
    return lax.bitcast_convert_type(word, jnp.int32)


def kernel(c, weight, index):
    B, C, X, Y = c.shape
    n_sup = weight.shape[0]
    XY = X * Y
    cpack = _pack_bf16_pair(c[:, 1], c[:, 0]).reshape(B * XY)
    wr = weight.reshape(n_sup, B, 2, XY)
    wgt = _pack_bf16_pair(wr[:, :, 0], wr[:, :, 1])
    ir = index.astype(jnp.int32).reshape(n_sup, B, 2, XY)
    idx = (ir[:, :, 0] << 16) | ir[:, :, 1]
    out = _spline_sc(cpack, wgt, idx, B=B, C=C, X=X, Y=Y)
    return out.reshape(B, C, X, Y)


# speedup vs baseline: 1.1841x; 1.1841x over previous
import dataclasses
import functools

import jax
import jax.numpy as jnp
from jax import lax
from jax.experimental import pallas as pl
from jax.experimental.pallas import tpu as pltpu
from jax.experimental.pallas import tpu_sc as plsc

L = 16


def _spline_sc(cpack, wgt, idx, *, B, C, X, Y, interpret=False):
    XY = X * Y
    NC, NS = 2, 16
    NW = NC * NS
    total = B * XY
    ppw = total // NW
    wpb = NW // B
    N = min(1024, ppw)
    NH = 0
    bpc = B // NC
    wpb_local = NS // bpc
    tab = bpc * XY
    nblk = ppw // N
    assert nblk % 2 == 0 and nblk >= 4
    mesh = plsc.VectorSubcoreMesh(
        core_axis_name="c", subcore_axis_name="s",
        num_cores=NC, num_subcores=NS)
    cp = pltpu.CompilerParams()
    if "needs_layout_passes" in pltpu.CompilerParams.__dataclass_fields__:
        cp = dataclasses.replace(cp, needs_layout_passes=False)

    @functools.partial(
        pl.kernel,
        out_type=jax.ShapeDtypeStruct((B, C, XY), jnp.float32),
        mesh=mesh,
        compiler_params=cp,
        interpret=interpret,
        scratch_types=[
            [pltpu.VMEM((8, N), jnp.float32)] * 2,
            [pltpu.VMEM((4, N), jnp.int32)] * 2,
            [[pltpu.VMEM((N,), jnp.int32)] * 16] * 2,
            [[pltpu.VMEM((N,), jnp.int32)] * 16] * 2,
            [pltpu.VMEM((C, N), jnp.float32)] * 2,
            pltpu.VMEM_SHARED((tab,), jnp.int32),
            pltpu.SemaphoreType.DMA,
            [pltpu.SemaphoreType.DMA] * 2,
            [pltpu.SemaphoreType.DMA] * 2,
            [pltpu.SemaphoreType.DMA] * 2,
        ],
    )
    def k(cpack_hbm, w_hbm, i_hbm, out_hbm,
          wbuf, ibuf, idxc, gbuf, obuf, ctab, semIn, semG, semGH, semO):
        sid = lax.axis_index("s")
        cid = lax.axis_index("c")
        b = cid * bpc + sid // wpb_local
        wofs = (sid % wpb_local) * ppw
        bofs = (sid // wpb_local) * XY

        @pl.when(sid == 0)
        def _stage():
            pltpu.sync_copy(cpack_hbm.at[pl.ds(cid * tab, tab)], ctab)

        plsc.subcore_barrier()

        def do_in(m, p):
            p0 = wofs + m * N

            @pl.loop(0, 8)
            def _ld(r):
                d = r // 4
                kk = r - d * 4
                pltpu.async_copy(
                    w_hbm.at[kk, b, d, pl.ds(p0, N)], wbuf[p].at[r], semIn)

            @pl.loop(0, 4)
            def _ldi(r):
                pltpu.async_copy(
                    i_hbm.at[r, b, pl.ds(p0, N)], ibuf[p].at[r], semIn)

        def do_build_fire(m, p):
            @pl.loop(0, 12)
            def _dw(r):
                pltpu.make_async_copy(
                    i_hbm.at[0, 0, pl.ds(0, N)], ibuf[p].at[0], semIn
                ).wait()

            @pl.loop(0, N, step=L)
            def _ix(j):
                sl = pl.ds(j, L)
                iqs = [ibuf[p][k, sl] for k in range(4)]
                iys = [iq & jnp.int32(0xFFFF) for iq in iqs]
                for k1 in range(4):
                    row = (iqs[k1] >> 16) * Y + bofs
                    for k2 in range(4):
                        idxc[p][k1 * 4 + k2][sl] = row + iys[k2]

            for cc in range(16):
                if cc < NH:
                    pltpu.async_copy(
                        cpack_hbm.at[idxc[p][cc]], gbuf[p][cc], semGH[p])
                else:
                    pltpu.async_copy(
                        ctab.at[idxc[p][cc]], gbuf[p][cc], semG[p])

        def do_acc(m, p):
            def _reclaim():
                @pl.loop(0, C)
                def _ow(ch):
                    pltpu.make_async_copy(
                        w_hbm.at[0, 0, 0, pl.ds(0, N)], obuf[p].at[0], semO[p]
                    ).wait()

            if isinstance(m, int):
                if m >= 2:
                    _reclaim()
            else:
                pl.when(m >= 2)(_reclaim)

            @pl.loop(0, NH)
            def _gwh(cc):
                pltpu.make_async_copy(
                    i_hbm.at[0, 0, pl.ds(0, N)], gbuf[p][0], semGH[p]
                ).wait()

            @pl.loop(0, 16 - NH)
            def _gw(cc):
                pltpu.make_async_copy(
                    i_hbm.at[0, 0, pl.ds(0, N)], gbuf[p][0], semG[p]
                ).wait()

            @pl.loop(0, N, step=L)
            def _acc(j):
                sl = pl.ds(j, L)
                w1 = [wbuf[p][k1, sl] for k1 in range(4)]
                w2 = [wbuf[p][4 + k2, sl] for k2 in range(4)]
                acc0 = jnp.zeros((L,), jnp.float32)
                acc1 = jnp.zeros((L,), jnp.float32)
                for k1 in range(4):
                    for k2 in range(4):
                        g = gbuf[p][k1 * 4 + k2][sl]
                        lo = plsc.bitcast(g << 16, jnp.float32)
                        hi = plsc.bitcast(g & jnp.int32(-65536), jnp.float32)
                        wp = w1[k1] * w2[k2]
                        acc0 = acc0 + wp * lo
                        acc1 = acc1 + wp * hi
                obuf[p][0, sl] = acc0
                obuf[p][1, sl] = acc1

            p0 = wofs + m * N

            @pl.loop(0, C)
            def _st(ch):
                pltpu.async_copy(
                    obuf[p].at[ch], out_hbm.at[b, ch, pl.ds(p0, N)], semO[p])

        do_in(0, 0)
        do_build_fire(0, 0)
        do_in(1, 1)

        @pl.loop(0, (nblk - 2) // 2)
        def _t(t):
            m = 2 * t + 1
            do_build_fire(m, 1)
            do_acc(m - 1, 0)
            do_in(m + 1, 0)
            do_build_fire(m + 1, 0)
            do_acc(m, 1)
            do_in(m + 2, 1)

        do_build_fire(nblk - 1, 1)
        do_acc(nblk - 2, 0)
        do_acc(nblk - 1, 1)

        for pp in range(2):
            @pl.loop(0, C)
            def _fw(i, pp=pp):
                pltpu.make_async_copy(
                    w_hbm.at[0, 0, 0, pl.ds(0, N)], obuf[0].at[0], semO[pp]
                ).wait()

    return k(cpack, wgt, idx)


def kernel(c, weight, index):
    B, C, X, Y = c.shape
    n_sup = weight.shape[0]
    XY = X * Y
    cb = c.astype(jnp.bfloat16)
    u = lax.bitcast_convert_type(cb, jnp.uint16).astype(jnp.uint32)
    word = (u[:, 1] << 16) | u[:, 0]
    cpack = lax.bitcast_convert_type(word, jnp.int32).reshape(B * XY)
    wgt = weight.reshape(n_sup, B, 2, XY)
    ir = index.astype(jnp.int32).reshape(n_sup, B, 2, XY)
    idx = (ir[:, :, 0] << 16) | ir[:, :, 1]
    out = _spline_sc(cpack, wgt, idx, B=B, C=C, X=X, Y=Y)
    return out.reshape(B, C, X, Y)


# speedup vs baseline: 1.2919x; 1.0911x over previous
import dataclasses
import functools

import jax
import jax.numpy as jnp
from jax import lax
from jax.experimental import pallas as pl
from jax.experimental.pallas import tpu as pltpu
from jax.experimental.pallas import tpu_sc as plsc

L = 16


def _spline_sc(cpack, wgt, idx, *, B, C, X, Y, interpret=False):
    XY = X * Y
    NC, NS = 2, 16
    NW = NC * NS
    total = B * XY
    ppw = total // NW
    wpb = NW // B
    N = min(512, ppw)
    NH = 0
    nblk = ppw // N
    assert nblk % 2 == 0 and nblk >= 4
    mesh = plsc.VectorSubcoreMesh(
        core_axis_name="c", subcore_axis_name="s",
        num_cores=NC, num_subcores=NS)
    cp = pltpu.CompilerParams()
    if "needs_layout_passes" in pltpu.CompilerParams.__dataclass_fields__:
        cp = dataclasses.replace(cp, needs_layout_passes=False)

    @functools.partial(
        pl.kernel,
        out_type=jax.ShapeDtypeStruct((B, C, XY), jnp.float32),
        mesh=mesh,
        compiler_params=cp,
        interpret=interpret,
        scratch_types=[
            [pltpu.VMEM((8, N), jnp.float32)] * 2,
            [pltpu.VMEM((8, N), jnp.int32)] * 2,
            [[pltpu.VMEM((N,), jnp.int32)] * 16] * 2,
            [[pltpu.VMEM((N,), jnp.int32)] * 16] * 2,
            [pltpu.VMEM((C, N), jnp.float32)] * 2,
            pltpu.VMEM_SHARED((total,), jnp.int32),
            pltpu.SemaphoreType.DMA,
            [pltpu.SemaphoreType.DMA] * 2,
            [pltpu.SemaphoreType.DMA] * 2,
            [pltpu.SemaphoreType.DMA] * 2,
        ],
    )
    def k(cpack_hbm, w_hbm, i_hbm, out_hbm,
          wbuf, ibuf, idxc, gbuf, obuf, ctab, semIn, semG, semGH, semO):
        sid = lax.axis_index("s")
        wid = sid * NC + lax.axis_index("c")
        b = wid // wpb
        wofs = (wid % wpb) * ppw
        bofs = b * XY

        @pl.when(sid == 0)
        def _stage():
            pltpu.sync_copy(cpack_hbm, ctab)

        plsc.subcore_barrier()

        def do_in(m, p):
            p0 = wofs + m * N

            @pl.loop(0, 8)
            def _ld(r):
                d = r // 4
                kk = r - d * 4
                pltpu.async_copy(
                    w_hbm.at[kk, b, d, pl.ds(p0, N)], wbuf[p].at[r], semIn)
                pltpu.async_copy(
                    i_hbm.at[kk, b, d, pl.ds(p0, N)], ibuf[p].at[r], semIn)

        def do_build_fire(m, p):
            @pl.loop(0, 16)
            def _dw(r):
                pltpu.make_async_copy(
                    i_hbm.at[0, 0, 0, pl.ds(0, N)], ibuf[p].at[0], semIn
                ).wait()

            @pl.loop(0, N, step=L)
            def _ix(j):
                sl = pl.ds(j, L)
                iys = [ibuf[p][4 + k2, sl] for k2 in range(4)]
                for k1 in range(4):
                    row = ibuf[p][k1, sl] * Y + bofs
                    for k2 in range(4):
                        idxc[p][k1 * 4 + k2][sl] = row + iys[k2]

            for cc in range(16):
                if cc < NH:
                    pltpu.async_copy(
                        cpack_hbm.at[idxc[p][cc]], gbuf[p][cc], semGH[p])
                else:
                    pltpu.async_copy(
                        ctab.at[idxc[p][cc]], gbuf[p][cc], semG[p])

        def do_acc(m, p):
            def _reclaim():
                @pl.loop(0, C)
                def _ow(ch):
                    pltpu.make_async_copy(
                        w_hbm.at[0, 0, 0, pl.ds(0, N)], obuf[p].at[0], semO[p]
                    ).wait()

            if isinstance(m, int):
                if m >= 2:
                    _reclaim()
            else:
                pl.when(m >= 2)(_reclaim)

            @pl.loop(0, NH)
            def _gwh(cc):
                pltpu.make_async_copy(
                    i_hbm.at[0, 0, 0, pl.ds(0, N)], gbuf[p][0], semGH[p]
                ).wait()

            @pl.loop(0, 16 - NH)
            def _gw(cc):
                pltpu.make_async_copy(
                    i_hbm.at[0, 0, 0, pl.ds(0, N)], gbuf[p][0], semG[p]
                ).wait()

            @pl.loop(0, N, step=L)
            def _acc(j):
                sl = pl.ds(j, L)
                w1 = [wbuf[p][k1, sl] for k1 in range(4)]
                w2 = [wbuf[p][4 + k2, sl] for k2 in range(4)]
                z = jnp.zeros((L,), jnp.float32)
                a0 = [z, z, z, z]
                a1 = [z, z, z, z]
                for k1 in range(4):
                    for k2 in range(4):
                        g = gbuf[p][k1 * 4 + k2][sl]
                        lo = plsc.bitcast(g << 16, jnp.float32)
                        hi = plsc.bitcast(g & jnp.int32(-65536), jnp.float32)
                        wp = w1[k1] * w2[k2]
                        a0[k2] = a0[k2] + wp * lo
                        a1[k2] = a1[k2] + wp * hi
                obuf[p][0, sl] = (a0[0] + a0[1]) + (a0[2] + a0[3])
                obuf[p][1, sl] = (a1[0] + a1[1]) + (a1[2] + a1[3])

            p0 = wofs + m * N

            @pl.loop(0, C)
            def _st(ch):
                pltpu.async_copy(
                    obuf[p].at[ch], out_hbm.at[b, ch, pl.ds(p0, N)], semO[p])

        do_in(0, 0)
        do_build_fire(0, 0)
        do_in(1, 1)

        @pl.loop(0, (nblk - 2) // 2)
        def _t(t):
            m = 2 * t + 1
            do_build_fire(m, 1)
            do_acc(m - 1, 0)
            do_in(m + 1, 0)
            do_build_fire(m + 1, 0)
            do_acc(m, 1)
            do_in(m + 2, 1)

        do_build_fire(nblk - 1, 1)
        do_acc(nblk - 2, 0)
        do_acc(nblk - 1, 1)

        for pp in range(2):
            @pl.loop(0, C)
            def _fw(i, pp=pp):
                pltpu.make_async_copy(
                    w_hbm.at[0, 0, 0, pl.ds(0, N)], obuf[0].at[0], semO[pp]
                ).wait()

    return k(cpack, wgt, idx)


def kernel(c, weight, index):
    B, C, X, Y = c.shape
    n_sup = weight.shape[0]
    XY = X * Y
    cb = c.astype(jnp.bfloat16)
    u = lax.bitcast_convert_type(cb, jnp.uint16).astype(jnp.uint32)
    word = (u[:, 1] << 16) | u[:, 0]
    cpack = lax.bitcast_convert_type(word, jnp.int32).reshape(B * XY)
    wgt = weight.reshape(n_sup, B, 2, XY)
    idx = index.astype(jnp.int32).reshape(n_sup, B, 2, XY)
    out = _spline_sc(cpack, wgt, idx, B=B, C=C, X=X, Y=Y)
    return out.reshape(B, C, X, Y)


# speedup vs baseline: 1.3185x; 1.0206x over previous
import dataclasses
import functools

import jax
import jax.numpy as jnp
from jax import lax
from jax.experimental import pallas as pl
from jax.experimental.pallas import tpu as pltpu
from jax.experimental.pallas import tpu_sc as plsc

L = 16


def _spline_sc(cpack, wgt, idx, *, B, C, X, Y, interpret=False):
    XY = X * Y
    NC, NS = 2, 16
    NW = NC * NS
    total = B * XY
    ppw = total // NW
    wpb = NW // B
    N = min(512, ppw)
    NH = 0
    nblk = ppw // N
    assert nblk % 2 == 0 and nblk >= 4
    mesh = plsc.VectorSubcoreMesh(
        core_axis_name="c", subcore_axis_name="s",
        num_cores=NC, num_subcores=NS)
    cp = pltpu.CompilerParams()
    if "needs_layout_passes" in pltpu.CompilerParams.__dataclass_fields__:
        cp = dataclasses.replace(cp, needs_layout_passes=False)

    @functools.partial(
        pl.kernel,
        out_type=jax.ShapeDtypeStruct((B, C, XY), jnp.float32),
        mesh=mesh,
        compiler_params=cp,
        interpret=interpret,
        scratch_types=[
            [pltpu.VMEM((8, N), jnp.float32)] * 2,
            [pltpu.VMEM((8, N), jnp.int32)] * 2,
            [pltpu.VMEM((16 * N,), jnp.int32)] * 2,
            [pltpu.VMEM((16 * N,), jnp.int32)] * 2,
            [pltpu.VMEM((C, N), jnp.float32)] * 2,
            pltpu.VMEM_SHARED((total,), jnp.int32),
            pltpu.SemaphoreType.DMA,
            [pltpu.SemaphoreType.DMA] * 2,
            [pltpu.SemaphoreType.DMA] * 2,
        ],
    )
    def k(cpack_hbm, w_hbm, i_hbm, out_hbm,
          wbuf, ibuf, idxc, gbuf, obuf, ctab, semIn, semG, semO):
        sid = lax.axis_index("s")
        wid = sid * NC + lax.axis_index("c")
        b = wid // wpb
        wofs = (wid % wpb) * ppw
        bofs = b * XY

        @pl.when(sid == 0)
        def _stage():
            pltpu.sync_copy(cpack_hbm, ctab)

        plsc.subcore_barrier()

        def do_in(m, p):
            p0 = wofs + m * N

            @pl.loop(0, 8)
            def _ld(r):
                d = r // 4
                kk = r - d * 4
                pltpu.async_copy(
                    w_hbm.at[kk, b, d, pl.ds(p0, N)], wbuf[p].at[r], semIn)
                pltpu.async_copy(
                    i_hbm.at[kk, b, d, pl.ds(p0, N)], ibuf[p].at[r], semIn)

        def do_build_fire(m, p):
            @pl.loop(0, 16)
            def _dw(r):
                pltpu.make_async_copy(
                    i_hbm.at[0, 0, 0, pl.ds(0, N)], ibuf[p].at[0], semIn
                ).wait()

            @pl.loop(0, N, step=L)
            def _ix(j):
                sl = pl.ds(j, L)
                iys = [ibuf[p][4 + k2, sl] for k2 in range(4)]
                for k1 in range(4):
                    row = ibuf[p][k1, sl] * Y + bofs
                    for k2 in range(4):
                        idxc[p][pl.ds((k1 * 4 + k2) * N + j, L)] = (
                            row + iys[k2])

            pltpu.async_copy(ctab.at[idxc[p]], gbuf[p], semG[p])

        def do_acc(m, p):
            def _reclaim():
                @pl.loop(0, C)
                def _ow(ch):
                    pltpu.make_async_copy(
                        w_hbm.at[0, 0, 0, pl.ds(0, N)], obuf[p].at[0], semO[p]
                    ).wait()

            if isinstance(m, int):
                if m >= 2:
                    _reclaim()
            else:
                pl.when(m >= 2)(_reclaim)

            pltpu.make_async_copy(
                i_hbm.at[0, 0, 0, pl.ds(0, 16 * N)], gbuf[p], semG[p]).wait()

            @pl.loop(0, N, step=L)
            def _acc(j):
                sl = pl.ds(j, L)
                w1 = [wbuf[p][k1, sl] for k1 in range(4)]
                w2 = [wbuf[p][4 + k2, sl] for k2 in range(4)]
                z = jnp.zeros((L,), jnp.float32)
                a0 = [z, z, z, z]
                a1 = [z, z, z, z]
                for k1 in range(4):
                    for k2 in range(4):
                        g = gbuf[p][pl.ds((k1 * 4 + k2) * N + j, L)]
                        lo = plsc.bitcast(g << 16, jnp.float32)
                        hi = plsc.bitcast(g & jnp.int32(-65536), jnp.float32)
                        wp = w1[k1] * w2[k2]
                        a0[k2] = a0[k2] + wp * lo
                        a1[k2] = a1[k2] + wp * hi
                obuf[p][0, sl] = (a0[0] + a0[1]) + (a0[2] + a0[3])
                obuf[p][1, sl] = (a1[0] + a1[1]) + (a1[2] + a1[3])

            p0 = wofs + m * N

            @pl.loop(0, C)
            def _st(ch):
                pltpu.async_copy(
                    obuf[p].at[ch], out_hbm.at[b, ch, pl.ds(p0, N)], semO[p])

        do_in(0, 0)
        do_build_fire(0, 0)
        do_in(1, 1)

        @pl.loop(0, (nblk - 2) // 2)
        def _t(t):
            m = 2 * t + 1
            do_build_fire(m, 1)
            do_acc(m - 1, 0)
            do_in(m + 1, 0)
            do_build_fire(m + 1, 0)
            do_acc(m, 1)
            do_in(m + 2, 1)

        do_build_fire(nblk - 1, 1)
        do_acc(nblk - 2, 0)
        do_acc(nblk - 1, 1)

        for pp in range(2):
            @pl.loop(0, C)
            def _fw(i, pp=pp):
                pltpu.make_async_copy(
                    w_hbm.at[0, 0, 0, pl.ds(0, N)], obuf[0].at[0], semO[pp]
                ).wait()

    return k(cpack, wgt, idx)


def kernel(c, weight, index):
    B, C, X, Y = c.shape
    n_sup = weight.shape[0]
    XY = X * Y
    cb = c.astype(jnp.bfloat16)
    u = lax.bitcast_convert_type(cb, jnp.uint16).astype(jnp.uint32)
    word = (u[:, 1] << 16) | u[:, 0]
    cpack = lax.bitcast_convert_type(word, jnp.int32).reshape(B * XY)
    wgt = weight.reshape(n_sup, B, 2, XY)
    idx = index.astype(jnp.int32).reshape(n_sup, B, 2, XY)
    out = _spline_sc(cpack, wgt, idx, B=B, C=C, X=X, Y=Y)
    return out.reshape(B, C, X, Y)
